# baseline (speedup 1.0000x reference)
import jax
import jax.numpy as jnp
import numpy as np
from jax import lax
from jax.experimental import pallas as pl
from jax.experimental.pallas import tpu as pltpu
from jax.experimental.pallas import tpu_sc as plsc

F = 26
E = 16
B = 16384
FIELD = 100000
EOD = F * E
MLP1 = 256
MLP2 = 128

NW = 32
BPW = B // NW
RPW = BPW * F
GSZ = 128
NG = RPW // GSZ
CHUNK = 13
NCH = NG // CHUNK
CROWS = CHUNK * GSZ

BLK = 1024
NBLK = B // BLK
RSQ = float(1.0 / np.sqrt(1.0 + 1e-5))


TOTAL = 2600000
TBCW = 3200
TBNC = TOTAL // TBCW
TBTAIL = TOTAL - TBNC * TBCW
TBUN = 8


def _tb_chunk(embF_hbm, oR_hbm, vpl, vout, iotC, sem, c0, cw):
    cps = []
    for e in range(E):
        cps.append(pltpu.async_copy(
            embF_hbm.at[pl.ds(e * TOTAL + c0, cw)],
            vpl.at[pl.ds(e * TBCW, cw)], sem))
    for c in cps:
        c.wait()

    def rows(g, carry2):
        r0 = g * TBUN
        for u in range(TBUN):
            r = r0 + u
            v = plsc.load_gather(vpl, [iotC + r])
            vout[pl.ds(pl.multiple_of(r * E, E), E)] = v
        return carry2

    lax.fori_loop(0, cw // TBUN, rows, 0)
    pltpu.sync_copy(vout.at[pl.ds(0, cw * E)], oR_hbm.at[pl.ds(c0 * E, cw * E)])


def _tb_body(embF_hbm, oR_hbm, vpl, vout, sem):
    wid = lax.axis_index("s") * 2 + lax.axis_index("c")
    iotC = lax.iota(jnp.int32, 16) * TBCW

    def chunk(g, carry):
        cid = g * NW + wid

        @pl.when(cid < TBNC)
        def _():
            _tb_chunk(embF_hbm, oR_hbm, vpl, vout, iotC, sem,
                      cid * TBCW, TBCW)

        return carry

    lax.fori_loop(0, -(-TBNC // NW), chunk, 0)

    @pl.when(wid == 5)
    def _():
        _tb_chunk(embF_hbm, oR_hbm, vpl, vout, iotC, sem,
                  TBNC * TBCW, TBTAIL)


def _sc_body(xi_hbm, emb_hbm, lin_hbm, e_out, lv_out, eidx, ebuf, lbuf,
             esem, lsem):
    wid = lax.axis_index("s") * 2 + lax.axis_index("c")
    base = wid * RPW
    pltpu.sync_copy(xi_hbm.at[wid], eidx)

    def chunk(i, carry):
        cps = []
        for j in range(CHUNK):
            r = i * CHUNK + j
            cps.append(pltpu.async_copy(
                emb_hbm.at[eidx.at[r]], ebuf.at[pl.ds(j * GSZ, GSZ)], esem))
            cps.append(pltpu.async_copy(
                lin_hbm.at[eidx.at[r]], lbuf.at[pl.ds(j * GSZ, GSZ)], lsem))
        for c in cps:
            c.wait()
        pltpu.sync_copy(ebuf, e_out.at[pl.ds(base + i * CROWS, CROWS)])
        pltpu.sync_copy(lbuf, lv_out.at[pl.ds(base + i * CROWS, CROWS)])
        return carry

    lax.fori_loop(0, NCH, chunk, 0)


T64 = 64
K1CW = 6400
K1NC = (TOTAL - T64) // K1CW
K1TAIL = TOTAL - T64 - K1NC * K1CW


def _tr_chunk(embT_hbm, linT_hbm, oe_hbm, ol_hbm, vpl, sem, c0, cw):
    cps = []
    for e in range(E):
        cps.append(pltpu.async_copy(
            embT_hbm.at[e, pl.ds(c0, cw)], vpl.at[pl.ds(e * K1CW, cw)], sem))
    cps.append(pltpu.async_copy(
        linT_hbm.at[0, pl.ds(c0, cw)], vpl.at[pl.ds(E * K1CW, cw)], sem))
    for c in cps:
        c.wait()
    cps = []
    for e in range(E):
        cps.append(pltpu.async_copy(
            vpl.at[pl.ds(e * K1CW, cw)], oe_hbm.at[pl.ds(e * TOTAL + c0, cw)],
            sem))
    cps.append(pltpu.async_copy(
        vpl.at[pl.ds(E * K1CW, cw)], ol_hbm.at[pl.ds(c0, cw)], sem))
    for c in cps:
        c.wait()


def _tr_body(embT_hbm, linT_hbm, tpl_hbm, tln_hbm, oe_hbm, ol_hbm, vpl, sem):
    wid = lax.axis_index("s") * 2 + lax.axis_index("c")

    def chunk(g, carry):
        cid = g * NW + wid

        @pl.when(cid < K1NC)
        def _():
            _tr_chunk(embT_hbm, linT_hbm, oe_hbm, ol_hbm, vpl, sem,
                      cid * K1CW, K1CW)

        return carry

    lax.fori_loop(0, -(-K1NC // NW), chunk, 0)

    @pl.when(wid == 1)
    def _():
        _tr_chunk(embT_hbm, linT_hbm, oe_hbm, ol_hbm, vpl, sem,
                  K1NC * K1CW, K1TAIL)

    @pl.when(wid == 3)
    def _():
        t0 = TOTAL - T64
        for e in range(E):
            pltpu.sync_copy(tpl_hbm.at[pl.ds(e * T64, T64)],
                            vpl.at[pl.ds(e * 128, T64)])
        pltpu.sync_copy(tln_hbm.at[pl.ds(0, T64)], vpl.at[pl.ds(E * 128, T64)])
        for e in range(E):
            pltpu.sync_copy(vpl.at[pl.ds(e * 128, T64)],
                            oe_hbm.at[pl.ds(e * TOTAL + t0, T64)])
        pltpu.sync_copy(vpl.at[pl.ds(E * 128, T64)], ol_hbm.at[pl.ds(t0, T64)])


def _tc_body(e_ref, lv_ref, s0_ref, b0_ref, cW_ref, cb_ref, M_ref, W1_ref,
             t1_ref, W2_ref, t2_ref, w3_ref, c0_ref, out_ref):
    f32 = jnp.float32
    e = e_ref[...] * s0_ref[...] + b0_ref[...]
    h = jnp.dot(e, cW_ref[...], preferred_element_type=f32) + cb_ref[...]
    lane = lax.broadcasted_iota(jnp.int32, h.shape, 1)
    h = jnp.where(lane < F, jnp.maximum(h, 0.0), -1e30)
    h = h - jnp.max(h, axis=1, keepdims=True)
    p = jnp.exp(h)
    wf = p / jnp.sum(p, axis=1, keepdims=True)
    ew = e * jnp.dot(wf, M_ref[...], preferred_element_type=f32)
    z = jnp.dot(ew, W1_ref[...], preferred_element_type=f32) + t1_ref[...]
    z = jnp.maximum(z, 0.0)
    z = jnp.dot(z, W2_ref[...], preferred_element_type=f32) + t2_ref[...]
    z = jnp.maximum(z, 0.0)
    acc = (jnp.sum(z * w3_ref[...], axis=1)
           + jnp.sum(lv_ref[...], axis=1) + c0_ref[0])
    out_ref[...] = jax.nn.sigmoid(acc)


_OFFSETS = np.arange(F, dtype=np.int32) * FIELD
_M = (np.arange(128)[:, None] == (np.arange(EOD)[None, :] // E)).astype(np.float32)


def kernel(x, emb_table, lin_table, lin_bias, bn0_g, bn0_b, ctrl_W, ctrl_b,
           ctrl_bn_g, ctrl_bn_b, W1, b1, bn1_g, bn1_b, W2, b2, bn2_g, bn2_b,
           W3, b3):
    xi = (x + jnp.asarray(_OFFSETS)[None, :]).reshape(NW, NG, GSZ)

    mesh1 = plsc.VectorSubcoreMesh(core_axis_name="c", subcore_axis_name="s")
    tr = pl.kernel(
        _tr_body,
        out_type=(jax.ShapeDtypeStruct((TOTAL * E,), jnp.float32),
                  jax.ShapeDtypeStruct((TOTAL,), jnp.float32)),
        mesh=mesh1,
        compiler_params=pltpu.CompilerParams(use_tc_tiling_on_sc=True),
        scratch_types=(
            pltpu.VMEM(((E + 1) * K1CW,), jnp.float32),
            pltpu.SemaphoreType.DMA,
        ),
    )
    tail_planes = emb_table[TOTAL - T64:].T.reshape(-1)
    tail_lin = lin_table[TOTAL - T64:].reshape(-1)
    embF, linF = tr(emb_table.T, lin_table.T, tail_planes, tail_lin)

    tb = pl.kernel(
        _tb_body,
        out_type=jax.ShapeDtypeStruct((TOTAL * E,), jnp.float32),
        mesh=plsc.VectorSubcoreMesh(core_axis_name="c", subcore_axis_name="s"),
        compiler_params=pltpu.CompilerParams(use_tc_tiling_on_sc=False,
                                             needs_layout_passes=False),
        scratch_types=(
            pltpu.VMEM((E * TBCW,), jnp.float32),
            pltpu.VMEM((TBCW * E,), jnp.float32),
            pltpu.SemaphoreType.DMA,
        ),
    )
    emb_tbl = tb(embF).reshape(TOTAL, E)

    mesh = plsc.VectorSubcoreMesh(core_axis_name="c", subcore_axis_name="s")
    sc = pl.kernel(
        _sc_body,
        out_type=(jax.ShapeDtypeStruct((B * F, E), jnp.float32),
                  jax.ShapeDtypeStruct((B * F,), jnp.float32)),
        mesh=mesh,
        compiler_params=pltpu.CompilerParams(use_tc_tiling_on_sc=False,
                                             needs_layout_passes=False),
        scratch_types=(
            pltpu.VMEM((NG, GSZ), jnp.int32),
            pltpu.VMEM((CROWS, E), jnp.float32),
            pltpu.VMEM((CROWS,), jnp.float32),
            pltpu.SemaphoreType.DMA,
            pltpu.SemaphoreType.DMA,
        ),
    )
    e_flat, lv_flat = sc(xi, emb_tbl, linF)
    e2 = e_flat.reshape(B, EOD)
    lv2 = lv_flat.reshape(B, F)

    s0 = jnp.repeat(bn0_g * RSQ, E).reshape(1, EOD)
    b0 = jnp.repeat(bn0_b, E).reshape(1, EOD)
    sctl = ctrl_bn_g * RSQ
    cW = jnp.zeros((EOD, 128), jnp.float32).at[:, :F].set(ctrl_W * sctl[None, :])
    cb = jnp.zeros((1, 128), jnp.float32).at[:, :F].set(ctrl_b * sctl + ctrl_bn_b)
    s1 = bn1_g * RSQ
    W1f = W1 * s1[None, :]
    t1 = (b1 * s1 + bn1_b).reshape(1, MLP1)
    s2 = bn2_g * RSQ
    W2f = W2 * s2[None, :]
    t2 = (b2 * s2 + bn2_b).reshape(1, MLP2)
    w3 = W3.reshape(1, MLP2)
    c0 = lin_bias + b3

    out2 = pl.pallas_call(
        _tc_body,
        grid=(NBLK,),
        in_specs=[
            pl.BlockSpec((BLK, EOD), lambda i: (i, 0)),
            pl.BlockSpec((BLK, F), lambda i: (i, 0)),
            pl.BlockSpec((1, EOD), lambda i: (0, 0)),
            pl.BlockSpec((1, EOD), lambda i: (0, 0)),
            pl.BlockSpec((EOD, 128), lambda i: (0, 0)),
            pl.BlockSpec((1, 128), lambda i: (0, 0)),
            pl.BlockSpec((128, EOD), lambda i: (0, 0)),
            pl.BlockSpec((EOD, MLP1), lambda i: (0, 0)),
            pl.BlockSpec((1, MLP1), lambda i: (0, 0)),
            pl.BlockSpec((MLP1, MLP2), lambda i: (0, 0)),
            pl.BlockSpec((1, MLP2), lambda i: (0, 0)),
            pl.BlockSpec((1, MLP2), lambda i: (0, 0)),
            pl.BlockSpec(memory_space=pltpu.SMEM),
        ],
        out_specs=pl.BlockSpec((BLK,), lambda i: (i,)),
        out_shape=jax.ShapeDtypeStruct((B,), jnp.float32),
    )(e2, lv2, s0, b0, cW, cb, jnp.asarray(_M), W1f, t1, W2f, t2, w3, c0)
    return out2


# speedup vs baseline: 2.1080x; 2.1080x over previous
import jax
import jax.numpy as jnp
import numpy as np
from jax import lax
from jax.experimental import pallas as pl
from jax.experimental.pallas import tpu as pltpu
from jax.experimental.pallas import tpu_sc as plsc

F = 26
E = 16
B = 16384
FIELD = 100000
EOD = F * E
MLP1 = 256
MLP2 = 128

NW = 32
BPW = B // NW
RPW = BPW * F
GSZ = 128
NG = RPW // GSZ
CHUNK = 13
NCH = NG // CHUNK
CROWS = CHUNK * GSZ

BLK = 1024
NBLK = B // BLK
RSQ = float(1.0 / np.sqrt(1.0 + 1e-5))


TOTAL = 2600000
CH2 = 8
NCH2 = NG // CH2
C2ROWS = CH2 * GSZ
NBUF = 4


def _sc_body(xi_hbm, embF_hbm, linF_hbm, e_out, lv_out,
             eidx, idxb, gbuf, ebuf, lbuf, esem, lsem):
    wid = lax.axis_index("s") * 2 + lax.axis_index("c")
    base = wid * RPW
    pltpu.sync_copy(xi_hbm.at[wid], eidx)
    iot = lax.iota(jnp.int32, 16)
    iotg = iot * GSZ

    def fire(r, s):
        for e in range(E):
            for g in range(GSZ // 16):
                idxb[pl.ds(s * E * GSZ + e * GSZ + g * 16, 16)] = (
                    eidx[r, pl.ds(g * 16, 16)] + e * TOTAL)
        cps = [pltpu.async_copy(
            embF_hbm.at[idxb.at[pl.ds(s * E * GSZ + e * GSZ, GSZ)]],
            gbuf.at[pl.ds(s * E * GSZ + e * GSZ, GSZ)], esem)
            for e in range(E)]
        return cps

    def assemble(jj, cps, lcp):
        for c in cps:
            c.wait()
        s = jj % NBUF
        for k in range(GSZ):
            v = plsc.load_gather(gbuf, [iotg + (s * E * GSZ + k)])
            ebuf[jj * GSZ + k] = v
        lcp.wait()

    def chunk(i, carry):
        handles = []
        for j in range(CH2):
            r = i * CH2 + j
            cps = fire(r, j % NBUF)
            lcp = pltpu.async_copy(
                linF_hbm.at[eidx.at[r]], lbuf.at[pl.ds(j * GSZ, GSZ)], lsem)
            handles.append((cps, lcp))
            if j >= NBUF - 1:
                jj = j - (NBUF - 1)
                assemble(jj, *handles[jj])
        for jj in range(CH2 - (NBUF - 1), CH2):
            assemble(jj, *handles[jj])
        pltpu.sync_copy(ebuf, e_out.at[pl.ds(base + i * C2ROWS, C2ROWS)])
        pltpu.sync_copy(lbuf, lv_out.at[pl.ds(base + i * C2ROWS, C2ROWS)])
        return carry

    lax.fori_loop(0, NCH2, chunk, 0)


T64 = 64
K1CW = 6400
K1NC = (TOTAL - T64) // K1CW
K1TAIL = TOTAL - T64 - K1NC * K1CW


def _tr_chunk(embT_hbm, linT_hbm, oe_hbm, ol_hbm, vpl, sem, c0, cw):
    cps = []
    for e in range(E):
        cps.append(pltpu.async_copy(
            embT_hbm.at[e, pl.ds(c0, cw)], vpl.at[pl.ds(e * K1CW, cw)], sem))
    cps.append(pltpu.async_copy(
        linT_hbm.at[0, pl.ds(c0, cw)], vpl.at[pl.ds(E * K1CW, cw)], sem))
    for c in cps:
        c.wait()
    cps = []
    for e in range(E):
        cps.append(pltpu.async_copy(
            vpl.at[pl.ds(e * K1CW, cw)], oe_hbm.at[pl.ds(e * TOTAL + c0, cw)],
            sem))
    cps.append(pltpu.async_copy(
        vpl.at[pl.ds(E * K1CW, cw)], ol_hbm.at[pl.ds(c0, cw)], sem))
    for c in cps:
        c.wait()


def _tr_body(embT_hbm, linT_hbm, tpl_hbm, tln_hbm, oe_hbm, ol_hbm, vpl, sem):
    wid = lax.axis_index("s") * 2 + lax.axis_index("c")

    def chunk(g, carry):
        cid = g * NW + wid

        @pl.when(cid < K1NC)
        def _():
            _tr_chunk(embT_hbm, linT_hbm, oe_hbm, ol_hbm, vpl, sem,
                      cid * K1CW, K1CW)

        return carry

    lax.fori_loop(0, -(-K1NC // NW), chunk, 0)

    @pl.when(wid == 1)
    def _():
        _tr_chunk(embT_hbm, linT_hbm, oe_hbm, ol_hbm, vpl, sem,
                  K1NC * K1CW, K1TAIL)

    @pl.when(wid == 3)
    def _():
        t0 = TOTAL - T64
        for e in range(E):
            pltpu.sync_copy(tpl_hbm.at[pl.ds(e * T64, T64)],
                            vpl.at[pl.ds(e * 128, T64)])
        pltpu.sync_copy(tln_hbm.at[pl.ds(0, T64)], vpl.at[pl.ds(E * 128, T64)])
        for e in range(E):
            pltpu.sync_copy(vpl.at[pl.ds(e * 128, T64)],
                            oe_hbm.at[pl.ds(e * TOTAL + t0, T64)])
        pltpu.sync_copy(vpl.at[pl.ds(E * 128, T64)], ol_hbm.at[pl.ds(t0, T64)])


def _tc_body(e_ref, lv_ref, s0_ref, b0_ref, cW_ref, cb_ref, M_ref, W1_ref,
             t1_ref, W2_ref, t2_ref, w3_ref, c0_ref, out_ref):
    f32 = jnp.float32
    e = e_ref[...] * s0_ref[...] + b0_ref[...]
    h = jnp.dot(e, cW_ref[...], preferred_element_type=f32) + cb_ref[...]
    lane = lax.broadcasted_iota(jnp.int32, h.shape, 1)
    h = jnp.where(lane < F, jnp.maximum(h, 0.0), -1e30)
    h = h - jnp.max(h, axis=1, keepdims=True)
    p = jnp.exp(h)
    wf = p / jnp.sum(p, axis=1, keepdims=True)
    ew = e * jnp.dot(wf, M_ref[...], preferred_element_type=f32)
    z = jnp.dot(ew, W1_ref[...], preferred_element_type=f32) + t1_ref[...]
    z = jnp.maximum(z, 0.0)
    z = jnp.dot(z, W2_ref[...], preferred_element_type=f32) + t2_ref[...]
    z = jnp.maximum(z, 0.0)
    acc = (jnp.sum(z * w3_ref[...], axis=1)
           + jnp.sum(lv_ref[...], axis=1) + c0_ref[0])
    out_ref[...] = jax.nn.sigmoid(acc)


_OFFSETS = np.arange(F, dtype=np.int32) * FIELD
_M = (np.arange(128)[:, None] == (np.arange(EOD)[None, :] // E)).astype(np.float32)


def kernel(x, emb_table, lin_table, lin_bias, bn0_g, bn0_b, ctrl_W, ctrl_b,
           ctrl_bn_g, ctrl_bn_b, W1, b1, bn1_g, bn1_b, W2, b2, bn2_g, bn2_b,
           W3, b3):
    xi = (x + jnp.asarray(_OFFSETS)[None, :]).reshape(NW, NG, GSZ)

    mesh1 = plsc.VectorSubcoreMesh(core_axis_name="c", subcore_axis_name="s")
    tr = pl.kernel(
        _tr_body,
        out_type=(jax.ShapeDtypeStruct((TOTAL * E,), jnp.float32),
                  jax.ShapeDtypeStruct((TOTAL,), jnp.float32)),
        mesh=mesh1,
        compiler_params=pltpu.CompilerParams(use_tc_tiling_on_sc=True),
        scratch_types=(
            pltpu.VMEM(((E + 1) * K1CW,), jnp.float32),
            pltpu.SemaphoreType.DMA,
        ),
    )
    tail_planes = emb_table[TOTAL - T64:].T.reshape(-1)
    tail_lin = lin_table[TOTAL - T64:].reshape(-1)
    embF, linF = tr(emb_table.T, lin_table.T, tail_planes, tail_lin)


    mesh = plsc.VectorSubcoreMesh(core_axis_name="c", subcore_axis_name="s")
    sc = pl.kernel(
        _sc_body,
        out_type=(jax.ShapeDtypeStruct((B * F, E), jnp.float32),
                  jax.ShapeDtypeStruct((B * F,), jnp.float32)),
        mesh=mesh,
        compiler_params=pltpu.CompilerParams(use_tc_tiling_on_sc=False,
                                             needs_layout_passes=False),
        scratch_types=(
            pltpu.VMEM((NG, GSZ), jnp.int32),
            pltpu.VMEM((NBUF * E * GSZ,), jnp.int32),
            pltpu.VMEM((NBUF * E * GSZ,), jnp.float32),
            pltpu.VMEM((C2ROWS, E), jnp.float32),
            pltpu.VMEM((C2ROWS,), jnp.float32),
            pltpu.SemaphoreType.DMA,
            pltpu.SemaphoreType.DMA,
        ),
    )
    e_flat, lv_flat = sc(xi, embF, linF)
    e2 = e_flat.reshape(B, EOD)
    lv2 = lv_flat.reshape(B, F)

    s0 = jnp.repeat(bn0_g * RSQ, E).reshape(1, EOD)
    b0 = jnp.repeat(bn0_b, E).reshape(1, EOD)
    sctl = ctrl_bn_g * RSQ
    cW = jnp.zeros((EOD, 128), jnp.float32).at[:, :F].set(ctrl_W * sctl[None, :])
    cb = jnp.zeros((1, 128), jnp.float32).at[:, :F].set(ctrl_b * sctl + ctrl_bn_b)
    s1 = bn1_g * RSQ
    W1f = W1 * s1[None, :]
    t1 = (b1 * s1 + bn1_b).reshape(1, MLP1)
    s2 = bn2_g * RSQ
    W2f = W2 * s2[None, :]
    t2 = (b2 * s2 + bn2_b).reshape(1, MLP2)
    w3 = W3.reshape(1, MLP2)
    c0 = lin_bias + b3

    out2 = pl.pallas_call(
        _tc_body,
        grid=(NBLK,),
        in_specs=[
            pl.BlockSpec((BLK, EOD), lambda i: (i, 0)),
            pl.BlockSpec((BLK, F), lambda i: (i, 0)),
            pl.BlockSpec((1, EOD), lambda i: (0, 0)),
            pl.BlockSpec((1, EOD), lambda i: (0, 0)),
            pl.BlockSpec((EOD, 128), lambda i: (0, 0)),
            pl.BlockSpec((1, 128), lambda i: (0, 0)),
            pl.BlockSpec((128, EOD), lambda i: (0, 0)),
            pl.BlockSpec((EOD, MLP1), lambda i: (0, 0)),
            pl.BlockSpec((1, MLP1), lambda i: (0, 0)),
            pl.BlockSpec((MLP1, MLP2), lambda i: (0, 0)),
            pl.BlockSpec((1, MLP2), lambda i: (0, 0)),
            pl.BlockSpec((1, MLP2), lambda i: (0, 0)),
            pl.BlockSpec(memory_space=pltpu.SMEM),
        ],
        out_specs=pl.BlockSpec((BLK,), lambda i: (i,)),
        out_shape=jax.ShapeDtypeStruct((B,), jnp.float32),
    )(e2, lv2, s0, b0, cW, cb, jnp.asarray(_M), W1f, t1, W2f, t2, w3, c0)
    return out2


# speedup vs baseline: 2.1357x; 1.0131x over previous
import jax
import jax.numpy as jnp
import numpy as np
from jax import lax
from jax.experimental import pallas as pl
from jax.experimental.pallas import tpu as pltpu
from jax.experimental.pallas import tpu_sc as plsc

F = 26
E = 16
B = 16384
FIELD = 100000
EOD = F * E
MLP1 = 256
MLP2 = 128

NW = 32
BPW = B // NW
RPW = BPW * F
GSZ = 128
NG = RPW // GSZ
CHUNK = 13
NCH = NG // CHUNK
CROWS = CHUNK * GSZ

BLK = 1024
NBLK = B // BLK
RSQ = float(1.0 / np.sqrt(1.0 + 1e-5))


TOTAL = 2600000
CH2 = 8
NCH2 = NG // CH2
C2ROWS = CH2 * GSZ
NBUF = 6


def _sc_body(xi_hbm, embF_hbm, linF_hbm, e_out, lv_out,
             eidx, idxb, gbuf, ebuf, lbuf, esem, lsem):
    wid = lax.axis_index("s") * 2 + lax.axis_index("c")
    base = wid * RPW
    pltpu.sync_copy(xi_hbm.at[wid], eidx)
    iot = lax.iota(jnp.int32, 16)
    iotg = iot * GSZ

    def fire(r, s):
        for e in range(E):
            for g in range(GSZ // 16):
                idxb[pl.ds(s * E * GSZ + e * GSZ + g * 16, 16)] = (
                    eidx[r, pl.ds(g * 16, 16)] + e * TOTAL)
        cps = [pltpu.async_copy(
            embF_hbm.at[idxb.at[pl.ds(s * E * GSZ + e * GSZ, GSZ)]],
            gbuf.at[pl.ds(s * E * GSZ + e * GSZ, GSZ)], esem)
            for e in range(E)]
        return cps

    def assemble(jj, cps, lcp):
        for c in cps:
            c.wait()
        s = jj % NBUF
        for k in range(GSZ):
            v = plsc.load_gather(gbuf, [iotg + (s * E * GSZ + k)])
            ebuf[jj * GSZ + k] = v
        lcp.wait()

    def chunk(i, carry):
        handles = []
        for j in range(CH2):
            r = i * CH2 + j
            cps = fire(r, j % NBUF)
            lcp = pltpu.async_copy(
                linF_hbm.at[eidx.at[r]], lbuf.at[pl.ds(j * GSZ, GSZ)], lsem)
            handles.append((cps, lcp))
            if j >= NBUF - 1:
                jj = j - (NBUF - 1)
                assemble(jj, *handles[jj])
        for jj in range(CH2 - (NBUF - 1), CH2):
            assemble(jj, *handles[jj])
        pltpu.sync_copy(ebuf, e_out.at[pl.ds(base + i * C2ROWS, C2ROWS)])
        pltpu.sync_copy(lbuf, lv_out.at[pl.ds(base + i * C2ROWS, C2ROWS)])
        return carry

    lax.fori_loop(0, NCH2, chunk, 0)


T64 = 64
K1CW = 6400
K1NC = (TOTAL - T64) // K1CW
K1TAIL = TOTAL - T64 - K1NC * K1CW


def _tr_chunk(embT_hbm, linT_hbm, oe_hbm, ol_hbm, vpl, sem, c0, cw):
    cps = []
    for e in range(E):
        cps.append(pltpu.async_copy(
            embT_hbm.at[e, pl.ds(c0, cw)], vpl.at[pl.ds(e * K1CW, cw)], sem))
    cps.append(pltpu.async_copy(
        linT_hbm.at[0, pl.ds(c0, cw)], vpl.at[pl.ds(E * K1CW, cw)], sem))
    for c in cps:
        c.wait()
    cps = []
    for e in range(E):
        cps.append(pltpu.async_copy(
            vpl.at[pl.ds(e * K1CW, cw)], oe_hbm.at[pl.ds(e * TOTAL + c0, cw)],
            sem))
    cps.append(pltpu.async_copy(
        vpl.at[pl.ds(E * K1CW, cw)], ol_hbm.at[pl.ds(c0, cw)], sem))
    for c in cps:
        c.wait()


def _tr_body(embT_hbm, linT_hbm, tpl_hbm, tln_hbm, oe_hbm, ol_hbm, vpl, sem):
    wid = lax.axis_index("s") * 2 + lax.axis_index("c")

    def chunk(g, carry):
        cid = g * NW + wid

        @pl.when(cid < K1NC)
        def _():
            _tr_chunk(embT_hbm, linT_hbm, oe_hbm, ol_hbm, vpl, sem,
                      cid * K1CW, K1CW)

        return carry

    lax.fori_loop(0, -(-K1NC // NW), chunk, 0)

    @pl.when(wid == 1)
    def _():
        _tr_chunk(embT_hbm, linT_hbm, oe_hbm, ol_hbm, vpl, sem,
                  K1NC * K1CW, K1TAIL)

    @pl.when(wid == 3)
    def _():
        t0 = TOTAL - T64
        for e in range(E):
            pltpu.sync_copy(tpl_hbm.at[pl.ds(e * T64, T64)],
                            vpl.at[pl.ds(e * 128, T64)])
        pltpu.sync_copy(tln_hbm.at[pl.ds(0, T64)], vpl.at[pl.ds(E * 128, T64)])
        for e in range(E):
            pltpu.sync_copy(vpl.at[pl.ds(e * 128, T64)],
                            oe_hbm.at[pl.ds(e * TOTAL + t0, T64)])
        pltpu.sync_copy(vpl.at[pl.ds(E * 128, T64)], ol_hbm.at[pl.ds(t0, T64)])


def _tc_body(e_ref, lv_ref, s0_ref, b0_ref, cW_ref, cb_ref, M_ref, W1_ref,
             t1_ref, W2_ref, t2_ref, w3_ref, c0_ref, out_ref):
    f32 = jnp.float32
    e = e_ref[...] * s0_ref[...] + b0_ref[...]
    h = jnp.dot(e, cW_ref[...], preferred_element_type=f32) + cb_ref[...]
    lane = lax.broadcasted_iota(jnp.int32, h.shape, 1)
    h = jnp.where(lane < F, jnp.maximum(h, 0.0), -1e30)
    h = h - jnp.max(h, axis=1, keepdims=True)
    p = jnp.exp(h)
    wf = p / jnp.sum(p, axis=1, keepdims=True)
    ew = e * jnp.dot(wf, M_ref[...], preferred_element_type=f32)
    z = jnp.dot(ew, W1_ref[...], preferred_element_type=f32) + t1_ref[...]
    z = jnp.maximum(z, 0.0)
    z = jnp.dot(z, W2_ref[...], preferred_element_type=f32) + t2_ref[...]
    z = jnp.maximum(z, 0.0)
    acc = (jnp.sum(z * w3_ref[...], axis=1)
           + jnp.sum(lv_ref[...], axis=1) + c0_ref[0])
    out_ref[...] = jax.nn.sigmoid(acc)


_OFFSETS = np.arange(F, dtype=np.int32) * FIELD
_M = (np.arange(128)[:, None] == (np.arange(EOD)[None, :] // E)).astype(np.float32)


def kernel(x, emb_table, lin_table, lin_bias, bn0_g, bn0_b, ctrl_W, ctrl_b,
           ctrl_bn_g, ctrl_bn_b, W1, b1, bn1_g, bn1_b, W2, b2, bn2_g, bn2_b,
           W3, b3):
    xi = (x + jnp.asarray(_OFFSETS)[None, :]).reshape(NW, NG, GSZ)

    mesh1 = plsc.VectorSubcoreMesh(core_axis_name="c", subcore_axis_name="s")
    tr = pl.kernel(
        _tr_body,
        out_type=(jax.ShapeDtypeStruct((TOTAL * E,), jnp.float32),
                  jax.ShapeDtypeStruct((TOTAL,), jnp.float32)),
        mesh=mesh1,
        compiler_params=pltpu.CompilerParams(use_tc_tiling_on_sc=True),
        scratch_types=(
            pltpu.VMEM(((E + 1) * K1CW,), jnp.float32),
            pltpu.SemaphoreType.DMA,
        ),
    )
    tail_planes = emb_table[TOTAL - T64:].T.reshape(-1)
    tail_lin = lin_table[TOTAL - T64:].reshape(-1)
    embF, linF = tr(emb_table.T, lin_table.T, tail_planes, tail_lin)


    mesh = plsc.VectorSubcoreMesh(core_axis_name="c", subcore_axis_name="s")
    sc = pl.kernel(
        _sc_body,
        out_type=(jax.ShapeDtypeStruct((B * F, E), jnp.float32),
                  jax.ShapeDtypeStruct((B * F,), jnp.float32)),
        mesh=mesh,
        compiler_params=pltpu.CompilerParams(use_tc_tiling_on_sc=False,
                                             needs_layout_passes=False),
        scratch_types=(
            pltpu.VMEM((NG, GSZ), jnp.int32),
            pltpu.VMEM((NBUF * E * GSZ,), jnp.int32),
            pltpu.VMEM((NBUF * E * GSZ,), jnp.float32),
            pltpu.VMEM((C2ROWS, E), jnp.float32),
            pltpu.VMEM((C2ROWS,), jnp.float32),
            pltpu.SemaphoreType.DMA,
            pltpu.SemaphoreType.DMA,
        ),
    )
    e_flat, lv_flat = sc(xi, embF, linF)
    e2 = e_flat.reshape(B, EOD)
    lv2 = lv_flat.reshape(B, F)

    s0 = jnp.repeat(bn0_g * RSQ, E).reshape(1, EOD)
    b0 = jnp.repeat(bn0_b, E).reshape(1, EOD)
    sctl = ctrl_bn_g * RSQ
    cW = jnp.zeros((EOD, 128), jnp.float32).at[:, :F].set(ctrl_W * sctl[None, :])
    cb = jnp.zeros((1, 128), jnp.float32).at[:, :F].set(ctrl_b * sctl + ctrl_bn_b)
    s1 = bn1_g * RSQ
    W1f = W1 * s1[None, :]
    t1 = (b1 * s1 + bn1_b).reshape(1, MLP1)
    s2 = bn2_g * RSQ
    W2f = W2 * s2[None, :]
    t2 = (b2 * s2 + bn2_b).reshape(1, MLP2)
    w3 = W3.reshape(1, MLP2)
    c0 = lin_bias + b3

    out2 = pl.pallas_call(
        _tc_body,
        grid=(NBLK,),
        in_specs=[
            pl.BlockSpec((BLK, EOD), lambda i: (i, 0)),
            pl.BlockSpec((BLK, F), lambda i: (i, 0)),
            pl.BlockSpec((1, EOD), lambda i: (0, 0)),
            pl.BlockSpec((1, EOD), lambda i: (0, 0)),
            pl.BlockSpec((EOD, 128), lambda i: (0, 0)),
            pl.BlockSpec((1, 128), lambda i: (0, 0)),
            pl.BlockSpec((128, EOD), lambda i: (0, 0)),
            pl.BlockSpec((EOD, MLP1), lambda i: (0, 0)),
            pl.BlockSpec((1, MLP1), lambda i: (0, 0)),
            pl.BlockSpec((MLP1, MLP2), lambda i: (0, 0)),
            pl.BlockSpec((1, MLP2), lambda i: (0, 0)),
            pl.BlockSpec((1, MLP2), lambda i: (0, 0)),
            pl.BlockSpec(memory_space=pltpu.SMEM),
        ],
        out_specs=pl.BlockSpec((BLK,), lambda i: (i,)),
        out_shape=jax.ShapeDtypeStruct((B,), jnp.float32),
    )(e2, lv2, s0, b0, cW, cb, jnp.asarray(_M), W1f, t1, W2f, t2, w3, c0)
    return out2
